# baseline (speedup 1.0000x reference)
import functools
import math

import jax
import jax.numpy as jnp
from jax import lax
from jax.experimental import pallas as pl
from jax.experimental.pallas import tpu as pltpu
from jax.experimental.pallas import tpu_sc as plsc

EMB = 64
SCALE = math.sqrt(EMB)
NC = 2
NS = 16
NW = NC * NS
LANES = 16


def _emb_body(n_chunks, chunk, tok_hbm, tab_hbm, out_hbm, idx_v, rows_v, sem_g):
    wid = lax.axis_index("s") * NC + lax.axis_index("c")
    per_w = n_chunks * chunk
    base = wid * per_w

    def do_chunk(i, _):
        off = base + i * chunk
        pltpu.sync_copy(tok_hbm.at[pl.ds(off, chunk)], idx_v)
        pltpu.async_copy(tab_hbm.at[idx_v], rows_v, sem_g).wait()

        def scale_row(r, _):
            for c in range(EMB // LANES):
                sl = pl.ds(c * LANES, LANES)
                rows_v[r, sl] = rows_v[r, sl] * SCALE
            return 0

        lax.fori_loop(0, chunk, scale_row, 0, unroll=2)
        pltpu.sync_copy(rows_v, out_hbm.at[pl.ds(off, chunk)])
        return 0

    lax.fori_loop(0, n_chunks, do_chunk, 0)


def kernel(tokens, table):
    b, s = tokens.shape
    n = b * s
    assert n % NW == 0
    per_w = n // NW
    chunk = 128
    assert per_w % chunk == 0
    n_chunks = per_w // chunk

    flat = tokens.reshape(n).astype(jnp.int32)

    mesh = plsc.VectorSubcoreMesh(
        core_axis_name="c", subcore_axis_name="s", num_cores=NC, num_subcores=NS
    )
    emb = pl.kernel(
        functools.partial(_emb_body, n_chunks, chunk),
        out_type=jax.ShapeDtypeStruct((n, EMB), jnp.float32),
        mesh=mesh,
        scratch_types=[
            pltpu.VMEM((chunk,), jnp.int32),
            pltpu.VMEM((chunk, EMB), jnp.float32),
            pltpu.SemaphoreType.DMA,
        ],
        compiler_params=pltpu.CompilerParams(use_tc_tiling_on_sc=False),
    )
    out = emb(flat, table)
    return out.reshape(b, s, EMB)


# speedup vs baseline: 1.2294x; 1.2294x over previous
import functools
import math

import jax
import jax.numpy as jnp
from jax import lax
from jax.experimental import pallas as pl
from jax.experimental.pallas import tpu as pltpu
from jax.experimental.pallas import tpu_sc as plsc

EMB = 64
SCALE = math.sqrt(EMB)
NC = 2
NS = 16
NW = NC * NS
LANES = 16
CHUNK = 128
NBUF = 4
LOOKAHEAD = 2


def _emb_body(n_chunks, tok_hbm, tab_hbm, out_hbm, idx_v, rows_v, sem_g, sem_w):
    wid = lax.axis_index("s") * NC + lax.axis_index("c")
    per_w = n_chunks * CHUNK
    base = wid * per_w

    pltpu.sync_copy(tok_hbm.at[wid], idx_v)

    def gather(j):
        pltpu.async_copy(tab_hbm.at[idx_v.at[j]], rows_v.at[j % NBUF], sem_g)

    def wait_gather(b):
        pltpu.make_async_copy(
            out_hbm.at[pl.ds(base, CHUNK)], rows_v.at[b], sem_g
        ).wait()

    def wait_write(b):
        pltpu.make_async_copy(
            rows_v.at[b], out_hbm.at[pl.ds(base, CHUNK)], sem_w
        ).wait()

    for j in range(LOOKAHEAD):
        gather(j)

    def step(j, _):
        b = j % NBUF

        @pl.when(j >= NBUF - LOOKAHEAD)
        def _():
            wait_write((j - (NBUF - LOOKAHEAD)) % NBUF)

        @pl.when(j + LOOKAHEAD < n_chunks)
        def _():
            gather(j + LOOKAHEAD)

        wait_gather(b)

        def scale_row(r, _):
            for c in range(EMB // LANES):
                sl = pl.ds(c * LANES, LANES)
                rows_v[b, r, sl] = rows_v[b, r, sl] * SCALE
            return 0

        lax.fori_loop(0, CHUNK, scale_row, 0, unroll=8)

        pltpu.async_copy(
            rows_v.at[b], out_hbm.at[pl.ds(base + j * CHUNK, CHUNK)], sem_w
        )
        return 0

    lax.fori_loop(0, n_chunks, step, 0)

    for k in range(NBUF - LOOKAHEAD):
        wait_write(k)


def kernel(tokens, table):
    b, s = tokens.shape
    n = b * s
    assert n % (NW * CHUNK) == 0
    per_w = n // NW
    n_chunks = per_w // CHUNK

    toks = tokens.reshape(NW, n_chunks, CHUNK).astype(jnp.int32)

    mesh = plsc.VectorSubcoreMesh(
        core_axis_name="c", subcore_axis_name="s", num_cores=NC, num_subcores=NS
    )
    emb = pl.kernel(
        functools.partial(_emb_body, n_chunks),
        out_type=jax.ShapeDtypeStruct((n, EMB), jnp.float32),
        mesh=mesh,
        scratch_types=[
            pltpu.VMEM((n_chunks, CHUNK), jnp.int32),
            pltpu.VMEM((NBUF, CHUNK, EMB), jnp.float32),
            pltpu.SemaphoreType.DMA,
            pltpu.SemaphoreType.DMA,
        ],
        compiler_params=pltpu.CompilerParams(use_tc_tiling_on_sc=False),
    )
    out = emb(toks, table)
    return out.reshape(b, s, EMB)
